# Initial kernel scaffold; baseline (speedup 1.0000x reference)
import jax
import jax.numpy as jnp
from jax.experimental import pallas as pl


def kernel(inputs_rgb, inputs_ir, targets_rgb, targets_ir, features_rgb, features_ir):
    raise NotImplementedError("write your pallas kernel here")


# speedup vs baseline: 3.9164x; 3.9164x over previous
import jax
import jax.numpy as jnp
from jax.experimental import pallas as pl
from jax.experimental.pallas import tpu as pltpu

B = 1024
D = 128
N = 100000
TILE_N = 2000
TEMP = 0.05
CAP = 1.0 / TEMP


def _cm_kernel(x_rgb_ref, x_ir_ref, t_rgb_ref, t_ir_ref, f_rgb_ref, f_ir_ref,
               out_rgb_ref, out_ir_ref,
               xn_rgb, xn_ir, s_rgb, s_ir, p_rgb, p_ir):
    c = pl.program_id(0)
    nc = pl.num_programs(0)

    @pl.when(c == 0)
    def _init():
        for x_ref, xn in ((x_rgb_ref, xn_rgb), (x_ir_ref, xn_ir)):
            x = x_ref[...]
            n = jnp.sqrt(jnp.sum(x * x, axis=1, keepdims=True))
            xn[...] = x / jnp.maximum(n, 1e-12)
        s_rgb[...] = jnp.zeros_like(s_rgb)
        s_ir[...] = jnp.zeros_like(s_ir)
        p_rgb[...] = jnp.zeros_like(p_rgb)
        p_ir[...] = jnp.zeros_like(p_ir)

    col = c * TILE_N + jax.lax.broadcasted_iota(jnp.int32, (1, TILE_N), 1)

    def bank(xn, t_ref, f_ref, s_acc, p_acc):
        logits = jax.lax.dot_general(
            xn[...], f_ref[...], (((1,), (1,)), ((), ())),
            preferred_element_type=jnp.float32) * (1.0 / TEMP)
        s_acc[...] += jnp.sum(jnp.exp(logits - CAP), axis=1, keepdims=True)
        mask = t_ref[...] == col
        p_acc[...] += jnp.sum(jnp.where(mask, logits, 0.0), axis=1,
                              keepdims=True)

    bank(xn_rgb, t_rgb_ref, f_rgb_ref, s_rgb, p_rgb)
    bank(xn_ir, t_ir_ref, f_ir_ref, s_ir, p_ir)

    @pl.when(c == nc - 1)
    def _fin():
        out_rgb_ref[...] = jnp.mean(
            CAP + jnp.log(s_rgb[...]) - p_rgb[...]).reshape(1, 1)
        out_ir_ref[...] = jnp.mean(
            CAP + jnp.log(s_ir[...]) - p_ir[...]).reshape(1, 1)


@jax.jit
def _run(x_rgb, x_ir, t_rgb, t_ir, f_rgb, f_ir):
    out = pl.pallas_call(
        _cm_kernel,
        grid=(N // TILE_N,),
        in_specs=[
            pl.BlockSpec((B, D), lambda c: (0, 0)),
            pl.BlockSpec((B, D), lambda c: (0, 0)),
            pl.BlockSpec((B, 1), lambda c: (0, 0)),
            pl.BlockSpec((B, 1), lambda c: (0, 0)),
            pl.BlockSpec((TILE_N, D), lambda c: (c, 0)),
            pl.BlockSpec((TILE_N, D), lambda c: (c, 0)),
        ],
        out_specs=[
            pl.BlockSpec((1, 1), lambda c: (0, 0)),
            pl.BlockSpec((1, 1), lambda c: (0, 0)),
        ],
        out_shape=[jax.ShapeDtypeStruct((1, 1), jnp.float32)] * 2,
        scratch_shapes=[
            pltpu.VMEM((B, D), jnp.float32),
            pltpu.VMEM((B, D), jnp.float32),
            pltpu.VMEM((B, 1), jnp.float32),
            pltpu.VMEM((B, 1), jnp.float32),
            pltpu.VMEM((B, 1), jnp.float32),
            pltpu.VMEM((B, 1), jnp.float32),
        ],
        compiler_params=pltpu.CompilerParams(
            dimension_semantics=("arbitrary",)),
    )(x_rgb, x_ir, t_rgb, t_ir, f_rgb, f_ir)
    return out[0][0, 0], out[1][0, 0]


def kernel(inputs_rgb, inputs_ir, targets_rgb, targets_ir,
           features_rgb, features_ir):
    return _run(inputs_rgb, inputs_ir,
                targets_rgb[:, None], targets_ir[:, None],
                features_rgb, features_ir)
